# Initial kernel scaffold; baseline (speedup 1.0000x reference)
import jax
import jax.numpy as jnp
from jax.experimental import pallas as pl


def kernel(x, l, center_img, center_skt):
    raise NotImplementedError("write your pallas kernel here")


# speedup vs baseline: 5.3703x; 5.3703x over previous
import functools

import jax
import jax.numpy as jnp
from jax import lax
from jax.experimental import pallas as pl
from jax.experimental.pallas import tpu as pltpu
from jax.experimental.pallas import tpu_sc as plsc

NCROPS = 2
NCLS = 100000
FDIM = 128
HALF = 64
NB = 16384
MOM = 0.9

_NTILE = 16
_CHUNK = NB // _NTILE
_NJ = _CHUNK // 128


def _sc_mesh():
    return plsc.VectorSubcoreMesh(core_axis_name="c", subcore_axis_name="s")


@functools.partial(
    pl.kernel,
    out_type=(
        jax.ShapeDtypeStruct((NB,), jnp.int32),
        jax.ShapeDtypeStruct((NB,), jnp.float32),
    ),
    mesh=_sc_mesh(),
    scratch_types=[
        pltpu.VMEM_SHARED((NCLS,), jnp.int32),
        pltpu.VMEM_SHARED((NB,), jnp.float32),
        pltpu.VMEM((_NJ, 128), jnp.int32),
        pltpu.VMEM((_NJ, 128), jnp.int32),
        pltpu.VMEM((_NJ, 128), jnp.int32),
        pltpu.VMEM((128,), jnp.float32),
        pltpu.VMEM((128,), jnp.float32),
    ],
)
def _k1(l_ref, rep_out, cnt_out, rep_s, cnt_s, lbuf, idbuf, repbuf, ones, zeros):
    cid = lax.axis_index("c")
    sid = lax.axis_index("s")
    base = sid * _CHUNK
    for j in range(_NJ):
        pltpu.sync_copy(l_ref.at[pl.ds(base + j * 128, 128)], lbuf.at[j])
    for k in range(8):
        ones[pl.ds(k * 16, 16)] = jnp.full((16,), 1.0, jnp.float32)
        zeros[pl.ds(k * 16, 16)] = jnp.zeros((16,), jnp.float32)
    for j in range(_NJ):
        for k in range(8):
            idbuf[j, pl.ds(k * 16, 16)] = (
                lax.iota(jnp.int32, 16) + (base + j * 128 + k * 16)
            )
    for j in range(_NJ):
        pltpu.sync_copy(zeros, cnt_s.at[pl.ds(base + j * 128, 128)])
    for j in range(_NJ):
        pltpu.sync_copy(idbuf.at[j], rep_s.at[lbuf.at[j]])
    plsc.subcore_barrier()
    for j in range(_NJ):
        pltpu.sync_copy(rep_s.at[lbuf.at[j]], repbuf.at[j])
    for j in range(_NJ):
        pltpu.sync_copy(ones, cnt_s.at[repbuf.at[j]], add=True)
    plsc.subcore_barrier()

    @pl.when(cid == 0)
    def _():
        for j in range(_NJ):
            pltpu.sync_copy(repbuf.at[j], rep_out.at[pl.ds(base + j * 128, 128)])
        pltpu.sync_copy(
            cnt_s.at[pl.ds(base, _CHUNK)], cnt_out.at[pl.ds(base, _CHUNK)]
        )


@functools.partial(
    pl.kernel,
    out_type=(
        jax.ShapeDtypeStruct((NB, FDIM), jnp.float32),
        jax.ShapeDtypeStruct((NB, FDIM), jnp.float32),
        jax.ShapeDtypeStruct((NB, FDIM), jnp.float32),
    ),
    mesh=_sc_mesh(),
    compiler_params=pltpu.CompilerParams(use_tc_tiling_on_sc=False),
    scratch_types=[
        pltpu.VMEM_SHARED((NB, HALF), jnp.float32),
        pltpu.VMEM((_NJ, 128), jnp.int32),
        pltpu.VMEM((128, HALF), jnp.float32),
        pltpu.VMEM((4, 128), jnp.int32),
        pltpu.VMEM((128, FDIM), jnp.float32),
    ],
)
def _k2(x_ref, rep_ref, l_ref, img_ref, skt_ref, acc_out, img_out, skt_out,
        acc_s, idxbuf, xbuf, lbuf, gbuf):
    cid = lax.axis_index("c")
    sid = lax.axis_index("s")
    row0 = sid * _CHUNK
    col0 = cid * HALF
    for r in range(128):
        for k in range(HALF // 16):
            xbuf[r, pl.ds(k * 16, 16)] = jnp.zeros((16,), jnp.float32)
    for j in range(_NJ):
        pltpu.sync_copy(xbuf, acc_s.at[pl.ds(row0 + j * 128, 128)])
    for j in range(_NJ):
        pltpu.sync_copy(rep_ref.at[pl.ds(row0 + j * 128, 128)], idxbuf.at[j])
    plsc.subcore_barrier()
    for crop in range(NCROPS):
        for j in range(_NJ):
            r = crop * NB + row0 + j * 128
            pltpu.sync_copy(x_ref.at[pl.ds(r, 128), pl.ds(col0, HALF)], xbuf)
            pltpu.sync_copy(xbuf, acc_s.at[idxbuf.at[j]], add=True)
    plsc.subcore_barrier()
    pltpu.sync_copy(
        acc_s.at[pl.ds(row0, _CHUNK)],
        acc_out.at[pl.ds(row0, _CHUNK), pl.ds(col0, HALF)],
    )
    gbase = (cid * _NTILE + sid) * 512
    for j in range(4):
        pltpu.sync_copy(l_ref.at[pl.ds(gbase + j * 128, 128)], lbuf.at[j])
    for j in range(4):
        pltpu.sync_copy(img_ref.at[lbuf.at[j]], gbuf)
        pltpu.sync_copy(gbuf, img_out.at[pl.ds(gbase + j * 128, 128)])
        pltpu.sync_copy(skt_ref.at[lbuf.at[j]], gbuf)
        pltpu.sync_copy(gbuf, skt_out.at[pl.ds(gbase + j * 128, 128)])


_BLK = 1024
_NBLK = NB // _BLK


def _k3_body(acc_ref, img_ref, skt_ref, cnt_ref, out_ref, s_ref):
    i = pl.program_id(0)

    @pl.when(i == 0)
    def _():
        s_ref[0] = 0.0
        s_ref[1] = 0.0

    k = cnt_ref[...]
    mf = 0.05 / jnp.maximum(k, 1.0)
    u = img_ref[...] * MOM + acc_ref[...] * mf
    nrm = lax.rsqrt(jnp.sum(u * u, axis=1, keepdims=True))
    dv = u * nrm - skt_ref[...]
    f = jnp.sum(dv * dv, axis=1, keepdims=True)
    valid = k > 0.0
    s_ref[0] += jnp.sum(jnp.where(valid, f, 0.0))
    s_ref[1] += jnp.sum(jnp.where(valid, 1.0, 0.0))

    @pl.when(i == _NBLK - 1)
    def _():
        out_ref[0, 0] = s_ref[0] / s_ref[1]


_k3 = pl.pallas_call(
    _k3_body,
    grid=(_NBLK,),
    in_specs=[
        pl.BlockSpec((_BLK, FDIM), lambda i: (i, 0)),
        pl.BlockSpec((_BLK, FDIM), lambda i: (i, 0)),
        pl.BlockSpec((_BLK, FDIM), lambda i: (i, 0)),
        pl.BlockSpec((_BLK, 1), lambda i: (i, 0)),
    ],
    out_specs=pl.BlockSpec(memory_space=pltpu.SMEM),
    out_shape=jax.ShapeDtypeStruct((1, 1), jnp.float32),
    scratch_shapes=[pltpu.SMEM((2,), jnp.float32)],
)


def kernel(x, l, center_img, center_skt):
    rep, cnt = _k1(l)
    accsum, img_g, skt_g = _k2(x, rep, l, center_img, center_skt)
    loss = _k3(accsum, img_g, skt_g, cnt.reshape(NB, 1))
    return loss[0, 0]


# speedup vs baseline: 6.6153x; 1.2318x over previous
import functools

import jax
import jax.numpy as jnp
from jax import lax
from jax.experimental import pallas as pl
from jax.experimental.pallas import tpu as pltpu
from jax.experimental.pallas import tpu_sc as plsc

NCROPS = 2
NCLS = 100000
FDIM = 128
HALF = 64
NB = 16384
MOM = 0.9

_NTILE = 16
_CHUNK = NB // _NTILE
_NJ = _CHUNK // 128


def _sc_mesh():
    return plsc.VectorSubcoreMesh(core_axis_name="c", subcore_axis_name="s")


@functools.partial(
    pl.kernel,
    out_type=(
        jax.ShapeDtypeStruct((NB,), jnp.int32),
        jax.ShapeDtypeStruct((NB,), jnp.float32),
    ),
    mesh=_sc_mesh(),
    scratch_types=[
        pltpu.VMEM_SHARED((NCLS,), jnp.int32),
        pltpu.VMEM_SHARED((NB,), jnp.float32),
        pltpu.VMEM((_NJ, 128), jnp.int32),
        pltpu.VMEM((_NJ, 128), jnp.int32),
        pltpu.VMEM((_NJ, 128), jnp.int32),
        pltpu.VMEM((128,), jnp.float32),
        pltpu.VMEM((128,), jnp.float32),
    ],
)
def _k1(l_ref, rep_out, cnt_out, rep_s, cnt_s, lbuf, idbuf, repbuf, ones, zeros):
    cid = lax.axis_index("c")
    sid = lax.axis_index("s")
    base = sid * _CHUNK
    for j in range(_NJ):
        pltpu.sync_copy(l_ref.at[pl.ds(base + j * 128, 128)], lbuf.at[j])
    for k in range(8):
        ones[pl.ds(k * 16, 16)] = jnp.full((16,), 1.0, jnp.float32)
        zeros[pl.ds(k * 16, 16)] = jnp.zeros((16,), jnp.float32)
    for j in range(_NJ):
        for k in range(8):
            idbuf[j, pl.ds(k * 16, 16)] = (
                lax.iota(jnp.int32, 16) + (base + j * 128 + k * 16)
            )
    for j in range(_NJ):
        pltpu.sync_copy(zeros, cnt_s.at[pl.ds(base + j * 128, 128)])
    for j in range(_NJ):
        pltpu.sync_copy(idbuf.at[j], rep_s.at[lbuf.at[j]])
    plsc.subcore_barrier()
    for j in range(_NJ):
        pltpu.sync_copy(rep_s.at[lbuf.at[j]], repbuf.at[j])
    for j in range(_NJ):
        pltpu.sync_copy(ones, cnt_s.at[repbuf.at[j]], add=True)
    plsc.subcore_barrier()

    @pl.when(cid == 0)
    def _():
        for j in range(_NJ):
            pltpu.sync_copy(repbuf.at[j], rep_out.at[pl.ds(base + j * 128, 128)])
        pltpu.sync_copy(
            cnt_s.at[pl.ds(base, _CHUNK)], cnt_out.at[pl.ds(base, _CHUNK)]
        )


@functools.partial(
    pl.kernel,
    out_type=(
        jax.ShapeDtypeStruct((NB, FDIM), jnp.float32),
        jax.ShapeDtypeStruct((NB, FDIM), jnp.float32),
        jax.ShapeDtypeStruct((NB, FDIM), jnp.float32),
    ),
    mesh=_sc_mesh(),
    compiler_params=pltpu.CompilerParams(use_tc_tiling_on_sc=False),
    scratch_types=[
        pltpu.VMEM_SHARED((NB, HALF), jnp.float32),
        pltpu.VMEM((_NJ, 128), jnp.int32),
        pltpu.VMEM((128, HALF), jnp.float32),
        pltpu.VMEM((128, HALF), jnp.float32),
        pltpu.VMEM((128, HALF), jnp.float32),
        pltpu.VMEM((4, 128), jnp.int32),
        pltpu.VMEM((128, FDIM), jnp.float32),
        pltpu.VMEM((128, FDIM), jnp.float32),
        pltpu.SemaphoreType.DMA,
        pltpu.SemaphoreType.DMA,
        pltpu.SemaphoreType.DMA,
        pltpu.SemaphoreType.DMA,
    ],
)
def _k2(x_ref, rep_ref, l_ref, img_ref, skt_ref, acc_out, img_out, skt_out,
        acc_s, idxbuf, xb0, xb1, zb, lbuf, gb0, gb1, sem0, sem1, semz, semg):
    cid = lax.axis_index("c")
    sid = lax.axis_index("s")
    row0 = sid * _CHUNK
    col0 = cid * HALF
    xbufs = (xb0, xb1)
    sems = (sem0, sem1)
    for j in range(_NJ):
        pltpu.async_copy(rep_ref.at[pl.ds(row0 + j * 128, 128)], idxbuf.at[j],
                         semz)
    for r in range(128):
        for k in range(HALF // 16):
            zb[r, pl.ds(k * 16, 16)] = jnp.zeros((16,), jnp.float32)
    zcps = [pltpu.async_copy(zb, acc_s.at[pl.ds(row0 + j * 128, 128)], semz)
            for j in range(_NJ)]
    for j in range(_NJ):
        pltpu.make_async_copy(rep_ref.at[pl.ds(row0 + j * 128, 128)],
                              idxbuf.at[j], semz).wait()
    for c in zcps:
        c.wait()
    plsc.subcore_barrier()
    nchunk = NCROPS * _NJ

    def _src(i):
        crop, j = divmod(i, _NJ)
        return x_ref.at[pl.ds(crop * NB + row0 + j * 128, 128),
                        pl.ds(col0, HALF)]

    pltpu.async_copy(_src(0), xbufs[0], sems[0])
    for i in range(nchunk):
        if i + 1 < nchunk:
            pltpu.async_copy(_src(i + 1), xbufs[(i + 1) % 2], sems[(i + 1) % 2])
        pltpu.make_async_copy(_src(i), xbufs[i % 2], sems[i % 2]).wait()
        pltpu.sync_copy(xbufs[i % 2], acc_s.at[idxbuf.at[i % _NJ]], add=True)
    plsc.subcore_barrier()
    acc_cp = pltpu.async_copy(
        acc_s.at[pl.ds(row0, _CHUNK)],
        acc_out.at[pl.ds(row0, _CHUNK), pl.ds(col0, HALF)],
        semz,
    )
    gbase = (cid * _NTILE + sid) * 512
    for j in range(4):
        pltpu.sync_copy(l_ref.at[pl.ds(gbase + j * 128, 128)], lbuf.at[j])
    gbufs = (gb0, gb1)

    def _gsrc(i):
        tab = img_ref if i < 4 else skt_ref
        return tab.at[lbuf.at[i % 4]]

    def _gdst(i):
        tab = img_out if i < 4 else skt_out
        return tab.at[pl.ds(gbase + (i % 4) * 128, 128)]

    pltpu.async_copy(_gsrc(0), gbufs[0], semg)
    for i in range(8):
        pltpu.make_async_copy(_gsrc(i), gbufs[i % 2], semg).wait()
        out_cp = pltpu.async_copy(gbufs[i % 2], _gdst(i), sems[i % 2])
        if i + 1 < 8:
            pltpu.async_copy(_gsrc(i + 1), gbufs[(i + 1) % 2], semg)
        if i + 1 < 8:
            out_cp.wait()
        else:
            out_cp.wait()
    acc_cp.wait()


_BLK = 1024
_NBLK = NB // _BLK


def _k3_body(acc_ref, img_ref, skt_ref, cnt_ref, out_ref, s_ref):
    i = pl.program_id(0)

    @pl.when(i == 0)
    def _():
        s_ref[0] = 0.0
        s_ref[1] = 0.0

    k = cnt_ref[...]
    mf = 0.05 / jnp.maximum(k, 1.0)
    u = img_ref[...] * MOM + acc_ref[...] * mf
    nrm = lax.rsqrt(jnp.sum(u * u, axis=1, keepdims=True))
    dv = u * nrm - skt_ref[...]
    f = jnp.sum(dv * dv, axis=1, keepdims=True)
    valid = k > 0.0
    s_ref[0] += jnp.sum(jnp.where(valid, f, 0.0))
    s_ref[1] += jnp.sum(jnp.where(valid, 1.0, 0.0))

    @pl.when(i == _NBLK - 1)
    def _():
        out_ref[0, 0] = s_ref[0] / s_ref[1]


_k3 = pl.pallas_call(
    _k3_body,
    grid=(_NBLK,),
    in_specs=[
        pl.BlockSpec((_BLK, FDIM), lambda i: (i, 0)),
        pl.BlockSpec((_BLK, FDIM), lambda i: (i, 0)),
        pl.BlockSpec((_BLK, FDIM), lambda i: (i, 0)),
        pl.BlockSpec((_BLK, 1), lambda i: (i, 0)),
    ],
    out_specs=pl.BlockSpec(memory_space=pltpu.SMEM),
    out_shape=jax.ShapeDtypeStruct((1, 1), jnp.float32),
    scratch_shapes=[pltpu.SMEM((2,), jnp.float32)],
)


def kernel(x, l, center_img, center_skt):
    rep, cnt = _k1(l)
    accsum, img_g, skt_g = _k2(x, rep, l, center_img, center_skt)
    loss = _k3(accsum, img_g, skt_g, cnt.reshape(NB, 1))
    return loss[0, 0]
